# baseline (speedup 1.0000x reference)
import functools

import jax
import jax.numpy as jnp
from jax import lax
from jax.experimental import pallas as pl
from jax.experimental.pallas import tpu as pltpu
from jax.experimental.pallas import tpu_sc as plsc

N = 10000
E = 320000
D = 128

NC = 2
NS = 16
NW = NC * NS

CB = 128
NCHUNK = 80
E_PAD = NCHUNK * CB * NW
CH_C0 = 160
CH_C1 = 0
N_PAD = 10240
ROWS_PER_TILE = N_PAD // NS
DEG_GRP = 8

_f32 = jnp.float32
_i32 = jnp.int32
_mesh = plsc.VectorSubcoreMesh(core_axis_name="c", subcore_axis_name="s")
_sc_params = pltpu.CompilerParams(needs_layout_passes=False)


@functools.partial(
    pl.kernel,
    out_type=jax.ShapeDtypeStruct((NW * N_PAD,), _f32),
    mesh=_mesh,
    compiler_params=_sc_params,
    scratch_types=[
        pltpu.VMEM((3 * DEG_GRP, CB), _i32),
        pltpu.VMEM((3 * DEG_GRP, CB), _i32),
        pltpu.VMEM((N_PAD,), _f32),
        pltpu.SemaphoreType.DMA,
        pltpu.SemaphoreType.DMA,
    ],
)
def _deg_kernel(epk_hbm, parts_hbm, eb0, eb1, degv, sg0, sg1):
    c = lax.axis_index("c")
    s = lax.axis_index("s")
    wid = s * NC + c
    base_r = wid * NCHUNK * 3

    zero16 = jnp.zeros((16,), _f32)

    def zbody(i, carry):
        degv[pl.ds(i * 16, 16)] = zero16
        return carry

    lax.fori_loop(0, N_PAD // 16, zbody, 0)

    def scatter_from(eb):
        for q in range(DEG_GRP):
            for k in range(CB // 16):
                idx = eb[3 * q + 1, pl.ds(k * 16, 16)]
                w = plsc.bitcast(eb[3 * q + 2, pl.ds(k * 16, 16)], _f32)
                plsc.addupdate_scatter(degv, [idx], w)

    ndeg2 = NCHUNK // (2 * DEG_GRP)
    GR = 3 * DEG_GRP
    pltpu.async_copy(epk_hbm.at[pl.ds(base_r, GR)], eb0, sg0)

    def body(t, carry):
        pltpu.async_copy(
            epk_hbm.at[pl.ds(base_r + (2 * t + 1) * GR, GR)], eb1, sg1)
        pltpu.make_async_copy(
            epk_hbm.at[pl.ds(base_r, GR)], eb0, sg0).wait()
        scatter_from(eb0)
        @pl.when(t < ndeg2 - 1)
        def _():
            pltpu.async_copy(
                epk_hbm.at[pl.ds(base_r + (2 * t + 2) * GR, GR)], eb0, sg0)
        pltpu.make_async_copy(
            epk_hbm.at[pl.ds(base_r, GR)], eb1, sg1).wait()
        scatter_from(eb1)
        return carry

    lax.fori_loop(0, ndeg2, body, 0)
    pltpu.sync_copy(degv, parts_hbm.at[pl.ds(wid * N_PAD, N_PAD)])


@functools.partial(
    pl.kernel,
    out_type=jax.ShapeDtypeStruct((NC, N_PAD, D), _f32),
    mesh=_mesh,
    compiler_params=_sc_params,
    scratch_types=[
        pltpu.VMEM((N_PAD,), _f32),
        pltpu.VMEM((3, CB), _i32),
        pltpu.VMEM((3, CB), _i32),
        pltpu.VMEM((CB,), _i32),
        pltpu.VMEM((CB,), _i32),
        pltpu.VMEM((CB,), _f32),
        pltpu.VMEM((CB, D), _f32),
        pltpu.VMEM((CB, D), _f32),
        pltpu.VMEM_SHARED((N_PAD, D), _f32),
        pltpu.SemaphoreType.DMA,
        pltpu.SemaphoreType.DMA,
        pltpu.SemaphoreType.DMA,
        pltpu.SemaphoreType.DMA,
        pltpu.SemaphoreType.DMA,
        pltpu.SemaphoreType.DMA,
    ],
)
def _msg_kernel(dis_hbm, h_hbm, epk_hbm, out_hbm,
                disv, eb0, eb1, cx0, cx1, nrmv, hr0, hr1, acc,
                sg0, sg1, ss0, ss1, se0, se1):
    c = lax.axis_index("c")
    s = lax.axis_index("s")
    nch = jnp.where(c == 0, CH_C0, CH_C1)
    base_l = jnp.where(c == 0, s * CH_C0, NS * CH_C0 + s * CH_C1)

    pltpu.sync_copy(dis_hbm, disv)

    zero16 = jnp.zeros((16,), _f32)

    def zrow(i, carry):
        for k in range(D // 16):
            hr0[i, pl.ds(k * 16, 16)] = zero16
        return carry

    lax.fori_loop(0, CB, zrow, 0)
    for j in range(ROWS_PER_TILE // CB):
        pltpu.sync_copy(hr0, acc.at[pl.ds(s * ROWS_PER_TILE + j * CB, CB)])
    plsc.subcore_barrier()

    def compute_norm(eb, cx):
        for k in range(CB // 16):
            ridx = eb[0, pl.ds(k * 16, 16)]
            cidx = eb[1, pl.ds(k * 16, 16)]
            cx[pl.ds(k * 16, 16)] = cidx
            w = plsc.bitcast(eb[2, pl.ds(k * 16, 16)], _f32)
            disr = plsc.load_gather(disv, [ridx])
            disc = plsc.load_gather(disv, [cidx])
            nrmv[pl.ds(k * 16, 16)] = w * disr * disc

    def scale_rows(hr):
        def scale(jj, carry2):
            for u in range(4):
                j = jj * 4 + u
                jvec = jnp.full((16,), j, dtype=_i32)
                sclr = plsc.load_gather(nrmv, [jvec])
                for k in range(D // 16):
                    hr[j, pl.ds(k * 16, 16)] = hr[j, pl.ds(k * 16, 16)] * sclr
            return carry2

        lax.fori_loop(0, CB // 4, scale, 0)

    def wait_scatter(hr, cx, ss):
        pltpu.make_async_copy(hr, acc.at[cx], ss).wait()

    def wait_gather(hr, eb, sg):
        pltpu.make_async_copy(h_hbm.at[eb.at[0]], hr, sg).wait()

    def wait_ebuf(eb, se, lin):
        pltpu.make_async_copy(epk_hbm.at[lin], eb, se).wait()

    niter = nch // 2

    @pl.when(niter > 0)
    def _():
        pltpu.sync_copy(epk_hbm.at[base_l], eb0)
        pltpu.async_copy(epk_hbm.at[base_l + 1], eb1, se1)
        pltpu.async_copy(h_hbm.at[eb0.at[0]], hr0, sg0)

    def body(j, carry):
        compute_norm(eb0, cx0)
        wait_ebuf(eb1, se1, base_l + 2 * j + 1)

        @pl.when(j > 0)
        def _():
            wait_scatter(hr1, cx1, ss1)
        pltpu.async_copy(h_hbm.at[eb1.at[0]], hr1, sg1)

        @pl.when(j < niter - 1)
        def _():
            pltpu.async_copy(epk_hbm.at[base_l + 2 * j + 2], eb0, se0)
        wait_gather(hr0, eb0, sg0)
        scale_rows(hr0)
        pltpu.async_copy(hr0, acc.at[cx0], ss0, add=True)

        compute_norm(eb1, cx1)

        @pl.when(j < niter - 1)
        def _():
            wait_ebuf(eb0, se0, base_l + 2 * j + 2)
        wait_scatter(hr0, cx0, ss0)

        @pl.when(j < niter - 1)
        def _():
            pltpu.async_copy(h_hbm.at[eb0.at[0]], hr0, sg0)
            pltpu.async_copy(epk_hbm.at[base_l + 2 * j + 3], eb1, se1)
        wait_gather(hr1, eb1, sg1)
        scale_rows(hr1)
        pltpu.async_copy(hr1, acc.at[cx1], ss1, add=True)
        return carry

    lax.fori_loop(0, niter, body, 0)

    @pl.when(niter > 0)
    def _():
        wait_scatter(hr1, cx1, ss1)
    plsc.subcore_barrier()
    pltpu.sync_copy(acc.at[pl.ds(s * ROWS_PER_TILE, ROWS_PER_TILE)],
                    out_hbm.at[c, pl.ds(s * ROWS_PER_TILE, ROWS_PER_TILE)])


def _tc_a_body(parts_ref, x_ref, w_ref, dis_ref, h_ref):
    deg = jnp.sum(parts_ref[...], axis=0)
    dis_ref[...] = jnp.where(deg > 0.0, lax.rsqrt(deg), 0.0)
    h_ref[...] = jnp.dot(x_ref[...], w_ref[...],
                         preferred_element_type=_f32)


_tc_a = pl.pallas_call(
    _tc_a_body,
    out_shape=(jax.ShapeDtypeStruct((N_PAD,), _f32),
               jax.ShapeDtypeStruct((N, D), _f32)),
)


def _tc_b_body(p_ref, b_ref, w_ref, o_ref):
    p = p_ref[0, :N, :] + p_ref[1, :N, :]
    g = jnp.maximum(p + b_ref[...][None, :], 0.0)
    o_ref[...] = jnp.dot(g, w_ref[...], preferred_element_type=_f32)


_tc_b = pl.pallas_call(
    _tc_b_body,
    out_shape=jax.ShapeDtypeStruct((N, D), _f32),
)


def _tc_c_body(p_ref, b_ref, x_ref, o_ref):
    p = p_ref[0, :N, :] + p_ref[1, :N, :]
    o_ref[...] = (jnp.maximum(p + b_ref[...][None, :], 0.0)
                  + x_ref[...])


_tc_c = pl.pallas_call(
    _tc_c_body,
    out_shape=jax.ShapeDtypeStruct((N, D), _f32),
)


def kernel(x, adj, edge_weights, W1, b1, W2, b2):
    row = adj[0].astype(_i32)
    col = adj[1].astype(_i32)
    padn = E_PAD - E
    row_p = jnp.pad(row, (0, padn)).reshape(NW * NCHUNK, CB)
    col_p = jnp.pad(col, (0, padn)).reshape(NW * NCHUNK, CB)
    wbits = lax.bitcast_convert_type(
        jnp.pad(edge_weights, (0, padn)), _i32).reshape(NW * NCHUNK, CB)
    epk = jnp.stack([row_p, col_p, wbits], axis=1)
    epk_flat = epk.reshape(NW * NCHUNK * 3, CB)

    deg_parts = _deg_kernel(epk_flat).reshape(NW, N_PAD)
    dis, h1 = _tc_a(deg_parts, x, W1)
    p1 = _msg_kernel(dis, h1, epk)
    h2 = _tc_b(p1, b1, W2)
    p2 = _msg_kernel(dis, h2, epk)
    out = _tc_c(p2, b2, x)
    return (out, adj, edge_weights)


# speedup vs baseline: 1.6172x; 1.6172x over previous
import functools

import jax
import jax.numpy as jnp
from jax import lax
from jax.experimental import pallas as pl
from jax.experimental.pallas import tpu as pltpu
from jax.experimental.pallas import tpu_sc as plsc

N = 10000
E = 320000
D = 128

NC = 2
NS = 16
NW = NC * NS

CB = 128
NCHUNK = 80
E_PAD = NCHUNK * CB * NW
CH_C0 = 152
CH_C1 = 8
N_PAD = 10240
ROWS_PER_TILE = N_PAD // NS
DEG_GRP = 8

_f32 = jnp.float32
_i32 = jnp.int32
_mesh = plsc.VectorSubcoreMesh(core_axis_name="c", subcore_axis_name="s")
_sc_params = pltpu.CompilerParams(needs_layout_passes=False)


@functools.partial(
    pl.kernel,
    out_type=jax.ShapeDtypeStruct((NW * N_PAD,), _f32),
    mesh=_mesh,
    compiler_params=_sc_params,
    scratch_types=[
        pltpu.VMEM((3 * DEG_GRP, CB), _i32),
        pltpu.VMEM((3 * DEG_GRP, CB), _i32),
        pltpu.VMEM((N_PAD,), _f32),
        pltpu.SemaphoreType.DMA,
        pltpu.SemaphoreType.DMA,
    ],
)
def _deg_kernel(epk_hbm, parts_hbm, eb0, eb1, degv, sg0, sg1):
    c = lax.axis_index("c")
    s = lax.axis_index("s")
    wid = s * NC + c
    base_r = wid * NCHUNK * 3

    zero16 = jnp.zeros((16,), _f32)

    def zbody(i, carry):
        degv[pl.ds(i * 16, 16)] = zero16
        return carry

    lax.fori_loop(0, N_PAD // 16, zbody, 0)

    def scatter_from(eb):
        for q in range(DEG_GRP):
            for k in range(CB // 16):
                idx = eb[3 * q + 1, pl.ds(k * 16, 16)]
                w = plsc.bitcast(eb[3 * q + 2, pl.ds(k * 16, 16)], _f32)
                plsc.addupdate_scatter(degv, [idx], w)

    ndeg2 = NCHUNK // (2 * DEG_GRP)
    GR = 3 * DEG_GRP
    pltpu.async_copy(epk_hbm.at[pl.ds(base_r, GR)], eb0, sg0)

    def body(t, carry):
        pltpu.async_copy(
            epk_hbm.at[pl.ds(base_r + (2 * t + 1) * GR, GR)], eb1, sg1)
        pltpu.make_async_copy(
            epk_hbm.at[pl.ds(base_r, GR)], eb0, sg0).wait()
        scatter_from(eb0)
        @pl.when(t < ndeg2 - 1)
        def _():
            pltpu.async_copy(
                epk_hbm.at[pl.ds(base_r + (2 * t + 2) * GR, GR)], eb0, sg0)
        pltpu.make_async_copy(
            epk_hbm.at[pl.ds(base_r, GR)], eb1, sg1).wait()
        scatter_from(eb1)
        return carry

    lax.fori_loop(0, ndeg2, body, 0)
    pltpu.sync_copy(degv, parts_hbm.at[pl.ds(wid * N_PAD, N_PAD)])


@functools.partial(
    pl.kernel,
    out_type=jax.ShapeDtypeStruct((NC, N_PAD, D), _f32),
    mesh=_mesh,
    compiler_params=_sc_params,
    scratch_types=[
        pltpu.VMEM((N_PAD,), _f32),
        pltpu.VMEM((3, CB), _i32),
        pltpu.VMEM((3, CB), _i32),
        pltpu.VMEM((CB,), _i32),
        pltpu.VMEM((CB,), _i32),
        pltpu.VMEM((CB,), _f32),
        pltpu.VMEM((CB, D), _f32),
        pltpu.VMEM((CB, D), _f32),
        pltpu.VMEM_SHARED((N_PAD, D), _f32),
        pltpu.SemaphoreType.DMA,
        pltpu.SemaphoreType.DMA,
        pltpu.SemaphoreType.DMA,
        pltpu.SemaphoreType.DMA,
        pltpu.SemaphoreType.DMA,
        pltpu.SemaphoreType.DMA,
    ],
)
def _msg_kernel(dis_hbm, h_hbm, epk_hbm, out_hbm,
                disv, eb0, eb1, cx0, cx1, nrmv, hr0, hr1, acc,
                sg0, sg1, ss0, ss1, se0, se1):
    c = lax.axis_index("c")
    s = lax.axis_index("s")
    nch = jnp.where(c == 0, CH_C0, CH_C1)
    base_l = jnp.where(c == 0, s * CH_C0, NS * CH_C0 + s * CH_C1)

    pltpu.sync_copy(dis_hbm, disv)

    zero16 = jnp.zeros((16,), _f32)

    def zrow(i, carry):
        for k in range(D // 16):
            hr0[i, pl.ds(k * 16, 16)] = zero16
        return carry

    lax.fori_loop(0, CB, zrow, 0)
    for j in range(ROWS_PER_TILE // CB):
        pltpu.sync_copy(hr0, acc.at[pl.ds(s * ROWS_PER_TILE + j * CB, CB)])
    plsc.subcore_barrier()

    def compute_norm(eb, cx):
        for k in range(CB // 16):
            ridx = eb[0, pl.ds(k * 16, 16)]
            cidx = eb[1, pl.ds(k * 16, 16)]
            cx[pl.ds(k * 16, 16)] = cidx
            w = plsc.bitcast(eb[2, pl.ds(k * 16, 16)], _f32)
            disr = plsc.load_gather(disv, [ridx])
            disc = plsc.load_gather(disv, [cidx])
            nrmv[pl.ds(k * 16, 16)] = w * disr * disc

    def scale_rows(hr):
        def scale(jj, carry2):
            for u in range(4):
                j = jj * 4 + u
                jvec = jnp.full((16,), j, dtype=_i32)
                sclr = plsc.load_gather(nrmv, [jvec])
                for k in range(D // 16):
                    hr[j, pl.ds(k * 16, 16)] = hr[j, pl.ds(k * 16, 16)] * sclr
            return carry2

        lax.fori_loop(0, CB // 4, scale, 0)

    def wait_scatter(hr, cx, ss):
        pltpu.make_async_copy(hr, acc.at[cx], ss).wait()

    def wait_gather(hr, eb, sg):
        pltpu.make_async_copy(h_hbm.at[eb.at[0]], hr, sg).wait()

    def wait_ebuf(eb, se, lin):
        pltpu.make_async_copy(epk_hbm.at[lin], eb, se).wait()

    niter = nch // 2

    @pl.when(niter > 0)
    def _():
        pltpu.sync_copy(epk_hbm.at[base_l], eb0)
        pltpu.async_copy(epk_hbm.at[base_l + 1], eb1, se1)
        pltpu.async_copy(h_hbm.at[eb0.at[0]], hr0, sg0)

    def body(j, carry):
        compute_norm(eb0, cx0)
        wait_ebuf(eb1, se1, base_l + 2 * j + 1)

        @pl.when(j > 0)
        def _():
            wait_scatter(hr1, cx1, ss1)
        pltpu.async_copy(h_hbm.at[eb1.at[0]], hr1, sg1)

        @pl.when(j < niter - 1)
        def _():
            pltpu.async_copy(epk_hbm.at[base_l + 2 * j + 2], eb0, se0)
        wait_gather(hr0, eb0, sg0)
        scale_rows(hr0)
        pltpu.async_copy(hr0, acc.at[cx0], ss0, add=True)

        compute_norm(eb1, cx1)

        @pl.when(j < niter - 1)
        def _():
            wait_ebuf(eb0, se0, base_l + 2 * j + 2)
        wait_scatter(hr0, cx0, ss0)

        @pl.when(j < niter - 1)
        def _():
            pltpu.async_copy(h_hbm.at[eb0.at[0]], hr0, sg0)
            pltpu.async_copy(epk_hbm.at[base_l + 2 * j + 3], eb1, se1)
        wait_gather(hr1, eb1, sg1)
        scale_rows(hr1)
        pltpu.async_copy(hr1, acc.at[cx1], ss1, add=True)
        return carry

    lax.fori_loop(0, niter, body, 0)

    @pl.when(niter > 0)
    def _():
        wait_scatter(hr1, cx1, ss1)
    plsc.subcore_barrier()
    pltpu.sync_copy(acc.at[pl.ds(s * ROWS_PER_TILE, ROWS_PER_TILE)],
                    out_hbm.at[c, pl.ds(s * ROWS_PER_TILE, ROWS_PER_TILE)])


def _tc_a_body(parts_ref, x_ref, w_ref, dis_ref, h_ref):
    deg = jnp.sum(parts_ref[...], axis=0)
    dis_ref[...] = jnp.where(deg > 0.0, lax.rsqrt(deg), 0.0)
    h_ref[...] = jnp.dot(x_ref[...], w_ref[...],
                         preferred_element_type=_f32)


_tc_a = pl.pallas_call(
    _tc_a_body,
    out_shape=(jax.ShapeDtypeStruct((N_PAD,), _f32),
               jax.ShapeDtypeStruct((N, D), _f32)),
)


def _tc_b_body(p_ref, b_ref, w_ref, o_ref):
    p = p_ref[0, :N, :] + p_ref[1, :N, :]
    g = jnp.maximum(p + b_ref[...][None, :], 0.0)
    o_ref[...] = jnp.dot(g, w_ref[...], preferred_element_type=_f32)


_tc_b = pl.pallas_call(
    _tc_b_body,
    out_shape=jax.ShapeDtypeStruct((N, D), _f32),
)


def _tc_c_body(p_ref, b_ref, x_ref, o_ref):
    p = p_ref[0, :N, :] + p_ref[1, :N, :]
    o_ref[...] = (jnp.maximum(p + b_ref[...][None, :], 0.0)
                  + x_ref[...])


_tc_c = pl.pallas_call(
    _tc_c_body,
    out_shape=jax.ShapeDtypeStruct((N, D), _f32),
)


def kernel(x, adj, edge_weights, W1, b1, W2, b2):
    row = adj[0].astype(_i32)
    col = adj[1].astype(_i32)
    padn = E_PAD - E
    row_p = jnp.pad(row, (0, padn)).reshape(NW * NCHUNK, CB)
    col_p = jnp.pad(col, (0, padn)).reshape(NW * NCHUNK, CB)
    wbits = lax.bitcast_convert_type(
        jnp.pad(edge_weights, (0, padn)), _i32).reshape(NW * NCHUNK, CB)
    epk = jnp.stack([row_p, col_p, wbits], axis=1)
    epk_flat = epk.reshape(NW * NCHUNK * 3, CB)

    deg_parts = _deg_kernel(epk_flat).reshape(NW, N_PAD)
    dis, h1 = _tc_a(deg_parts, x, W1)
    p1 = _msg_kernel(dis, h1, epk)
    h2 = _tc_b(p1, b1, W2)
    p2 = _msg_kernel(dis, h2, epk)
    out = _tc_c(p2, b2, x)
    return (out, adj, edge_weights)
